# baseline (speedup 1.0000x reference)
import functools

import jax
import jax.numpy as jnp
from jax import lax
from jax.experimental import pallas as pl
from jax.experimental.pallas import tpu as pltpu
from jax.experimental.pallas import tpu_sc as plsc

N = 50000
D = 256
G = 256
NC = 2
NS = 16
DC = D // NC
NG16 = DC // 16
SBR = 48
CHUNK = (N // NS + 7) // 8 * 8
NFULL = (N // NS - 8) // SBR + 1
BLK = 2000


def _sc_stats_body(x_hbm, batch_hbm, o1, o2,
                   p_sh, s1v, s2v, xv, idxv, semx):
    cid = lax.axis_index("c")
    sid = lax.axis_index("s")
    ccol = cid * DC
    zeros16 = jnp.zeros((16,), jnp.float32)

    def _zero(r, _):
        for c in range(NG16):
            s1v[r, pl.ds(c * 16, 16)] = zeros16
            s2v[r, pl.ds(c * 16, 16)] = zeros16
        return 0

    lax.fori_loop(0, G, _zero, 0)

    start = sid * N // NS // 8 * 8
    end = (sid + 1) * N // NS // 8 * 8
    nrows = end - start

    pltpu.sync_copy(batch_hbm.at[pl.ds(start, CHUNK)], idxv)

    def _start_x(k, b):
        return pltpu.async_copy(
            x_hbm.at[pl.ds(start + k * SBR, SBR), pl.ds(ccol, DC)],
            xv.at[b], semx.at[b])

    def _process(b, ioff, lo):
        @plsc.parallel_loop(0, SBR // 16, 1, unroll=2)
        def _row16(i):
            idvec = idxv[pl.ds(ioff + i * 16, 16)]
            for l in range(16):
                r = i * 16 + l
                g = idvec[l]
                m = (r >= lo).astype(jnp.float32)
                for c in range(NG16):
                    v = xv[b, r, pl.ds(c * 16, 16)] * m
                    plsc.addupdate(s1v.at[g, pl.ds(c * 16, 16)], v)
                    plsc.addupdate(s2v.at[g, pl.ds(c * 16, 16)], v * v)

    _start_x(0, 0)
    _start_x(1, 1)

    @pl.loop(0, NFULL // 2 * 2, step=2)
    def _ring(k):
        for b in range(2):
            kk = k + b
            pltpu.make_async_copy(
                x_hbm.at[pl.ds(start + kk * SBR, SBR), pl.ds(ccol, DC)],
                xv.at[b], semx.at[b]).wait()
            _process(b, kk * SBR, 0)

            @pl.when(kk + 2 < NFULL)
            def _next():
                _start_x(kk + 2, b)

    if NFULL % 2:
        kk = NFULL - 1
        pltpu.make_async_copy(
            x_hbm.at[pl.ds(start + kk * SBR, SBR), pl.ds(ccol, DC)],
            xv.at[0], semx.at[0]).wait()
        _process(0, kk * SBR, 0)

    rem = nrows - NFULL * SBR
    pltpu.sync_copy(x_hbm.at[pl.ds(end - SBR, SBR), pl.ds(ccol, DC)], xv.at[0])
    _process(0, nrows - SBR, SBR - rem)

    GH = G // 2
    STRIP = GH // NS

    def _phase(src_v, glo, out_hbm):
        pltpu.sync_copy(src_v.at[pl.ds(glo, GH)], p_sh.at[sid])
        plsc.subcore_barrier()
        rlo = sid * STRIP
        pltpu.sync_copy(p_sh.at[0, pl.ds(rlo, STRIP)],
                        xv.at[0, pl.ds(0, STRIP)])

        def _racc(j, _):
            pltpu.sync_copy(p_sh.at[j, pl.ds(rlo, STRIP)],
                            xv.at[0, pl.ds(STRIP, STRIP)])

            def _radd(r, _):
                for c in range(NG16):
                    plsc.addupdate(xv.at[0, r, pl.ds(c * 16, 16)],
                                   xv[0, STRIP + r, pl.ds(c * 16, 16)])
                return 0

            lax.fori_loop(0, STRIP, _radd, 0)
            return 0

        lax.fori_loop(1, NS, _racc, 0)
        pltpu.sync_copy(xv.at[0, pl.ds(0, STRIP)],
                        out_hbm.at[pl.ds(glo + rlo, STRIP), pl.ds(ccol, DC)])
        plsc.subcore_barrier()

    _phase(s1v, 0, o1)
    _phase(s1v, GH, o1)
    _phase(s2v, 0, o2)
    _phase(s2v, GH, o2)


def _finalize_body(batch_ref, o1_ref, o2_ref, w_ref, b_ref, s_ref,
                   rb_ref, cnt_ref):
    i = pl.program_id(0)
    nb = pl.num_programs(0)

    @pl.when(i == 0)
    def _init():
        cnt_ref[...] = jnp.zeros_like(cnt_ref)

    bb = batch_ref[0, 0, :]
    onehot = (bb[:, None] == lax.broadcasted_iota(jnp.int32, (BLK, G), 1)
              ).astype(jnp.float32)
    cnt_ref[...] += jnp.sum(onehot, axis=0)[None, :]

    @pl.when(i == nb - 1)
    def _fin():
        s1 = o1_ref[...]
        s2 = o2_ref[...]
        cnt = cnt_ref[0, :][:, None]
        denom = jnp.maximum(cnt, 1.0)
        a = (s1 / denom) * s_ref[...]
        var = (s2 - 2.0 * a * s1 + cnt * a * a) / denom
        r = w_ref[...] * lax.rsqrt(var + 1e-8)
        bp = b_ref[...] - a * r
        rb_ref[...] = jnp.concatenate([r, bp], axis=1)


def _norm_body(x_ref, batch_ref, rb_ref, out_ref):
    bb = batch_ref[0, 0, :]
    onehot = (bb[:, None] == lax.broadcasted_iota(jnp.int32, (BLK, G), 1)
              ).astype(jnp.float32)
    g = lax.dot_general(onehot, rb_ref[...], (((1,), (0,)), ((), ())),
                        preferred_element_type=jnp.float32)
    x = x_ref[...]
    out_ref[...] = x * g[:, :D] + g[:, D:]


@jax.jit
def kernel(node_emb, weight, bias, scale, batch):
    n, d = node_emb.shape
    nb = n // BLK
    batch_i = batch.astype(jnp.int32)

    mesh = plsc.VectorSubcoreMesh(core_axis_name="c", subcore_axis_name="s")
    o1, o2 = pl.kernel(
        _sc_stats_body,
        out_type=(
            jax.ShapeDtypeStruct((G, D), jnp.float32),
            jax.ShapeDtypeStruct((G, D), jnp.float32),
        ),
        mesh=mesh,
        scratch_types=[
            pltpu.VMEM_SHARED((NS, G // 2, DC), jnp.float32),
            pltpu.VMEM((G, DC), jnp.float32),
            pltpu.VMEM((G, DC), jnp.float32),
            pltpu.VMEM((2, SBR, DC), jnp.float32),
            pltpu.VMEM((CHUNK,), jnp.int32),
            pltpu.SemaphoreType.DMA((2,)),
        ],
    )(node_emb, batch_i)

    w2 = weight.reshape(1, d)
    b2 = bias.reshape(1, d)
    s2 = scale.reshape(1, d)
    batch3 = batch_i.reshape(nb, 1, BLK)
    rb = pl.pallas_call(
        _finalize_body,
        grid=(nb,),
        in_specs=[
            pl.BlockSpec((1, 1, BLK), lambda i: (i, 0, 0)),
            pl.BlockSpec((G, d), lambda i: (0, 0)),
            pl.BlockSpec((G, d), lambda i: (0, 0)),
            pl.BlockSpec((1, d), lambda i: (0, 0)),
            pl.BlockSpec((1, d), lambda i: (0, 0)),
            pl.BlockSpec((1, d), lambda i: (0, 0)),
        ],
        out_specs=pl.BlockSpec((G, 2 * d), lambda i: (0, 0)),
        out_shape=jax.ShapeDtypeStruct((G, 2 * d), jnp.float32),
        scratch_shapes=[pltpu.VMEM((1, G), jnp.float32)],
    )(batch3, o1, o2, w2, b2, s2)
    out = pl.pallas_call(
        _norm_body,
        grid=(nb,),
        in_specs=[
            pl.BlockSpec((BLK, d), lambda i: (i, 0)),
            pl.BlockSpec((1, 1, BLK), lambda i: (i, 0, 0)),
            pl.BlockSpec((G, 2 * d), lambda i: (0, 0)),
        ],
        out_specs=pl.BlockSpec((BLK, d), lambda i: (i, 0)),
        out_shape=jax.ShapeDtypeStruct((n, d), jnp.float32),
    )(node_emb, batch3, rb)
    return out


# speedup vs baseline: 1.7368x; 1.7368x over previous
import functools

import jax
import jax.numpy as jnp
from jax import lax
from jax.experimental import pallas as pl
from jax.experimental.pallas import tpu as pltpu
from jax.experimental.pallas import tpu_sc as plsc

N = 50000
D = 256
G = 256
NC = 2
NS = 16
DC = D // NC
NG16 = DC // 16
SBR = 48
CHUNK = (N // NS + 7) // 8 * 8
NFULL = (N // NS - 8) // SBR + 1
BLK = 2000


def _sc_stats_body(x_hbm, batch_hbm, o1, o2,
                   p_sh, s1v, s2v, xv, idxv, gref, semx):
    cid = lax.axis_index("c")
    sid = lax.axis_index("s")
    ccol = cid * DC
    zeros16 = jnp.zeros((16,), jnp.float32)

    def _zero(r, _):
        for c in range(NG16):
            s1v[r, pl.ds(c * 16, 16)] = zeros16
            s2v[r, pl.ds(c * 16, 16)] = zeros16
        return 0

    lax.fori_loop(0, G, _zero, 0)

    start = sid * N // NS // 8 * 8
    end = (sid + 1) * N // NS // 8 * 8
    nrows = end - start

    pltpu.sync_copy(batch_hbm.at[pl.ds(start, CHUNK)], idxv)

    def _start_x(k, b):
        return pltpu.async_copy(
            x_hbm.at[pl.ds(start + k * SBR, SBR), pl.ds(ccol, DC)],
            xv.at[b], semx.at[b])

    def _process(b, ioff, lo):
        def _row16(i, _):
            idvec = idxv[pl.ds(ioff + i * 16, 16)]
            g0 = idvec[0]
            g15 = idvec[15]
            a1 = [jnp.zeros((16,), jnp.float32)] * NG16
            a2 = [jnp.zeros((16,), jnp.float32)] * NG16
            for l in range(16):
                r = i * 16 + l
                ma = ((r >= lo) & (idvec[l] == g0)).astype(jnp.float32)
                for c in range(NG16):
                    v = xv[b, r, pl.ds(c * 16, 16)]
                    a1[c] = a1[c] + v * ma
                    a2[c] = a2[c] + (v * v) * ma
            for c in range(NG16):
                plsc.addupdate(s1v.at[g0, pl.ds(c * 16, 16)], a1[c])
                plsc.addupdate(s2v.at[g0, pl.ds(c * 16, 16)], a2[c])

            @pl.when(g15 != g0)
            def _slow():
                idv = idxv[pl.ds(ioff + i * 16, 16)]
                gg0 = idv[0]
                for l in range(16):
                    r = i * 16 + l
                    g = idv[l]
                    mb = ((r >= lo) & (g != gg0)).astype(jnp.float32)
                    for c in range(NG16):
                        v = xv[b, r, pl.ds(c * 16, 16)] * mb
                        plsc.addupdate(s1v.at[g, pl.ds(c * 16, 16)], v)
                        plsc.addupdate(s2v.at[g, pl.ds(c * 16, 16)], v * v)

            return 0

        lax.fori_loop(0, SBR // 16, _row16, 0)

    _start_x(0, 0)
    _start_x(1, 1)

    @pl.loop(0, NFULL // 2 * 2, step=2)
    def _ring(k):
        for b in range(2):
            kk = k + b
            pltpu.make_async_copy(
                x_hbm.at[pl.ds(start + kk * SBR, SBR), pl.ds(ccol, DC)],
                xv.at[b], semx.at[b]).wait()
            _process(b, kk * SBR, 0)

            @pl.when(kk + 2 < NFULL)
            def _next():
                _start_x(kk + 2, b)

    if NFULL % 2:
        kk = NFULL - 1
        pltpu.make_async_copy(
            x_hbm.at[pl.ds(start + kk * SBR, SBR), pl.ds(ccol, DC)],
            xv.at[0], semx.at[0]).wait()
        _process(0, kk * SBR, 0)

    rem = nrows - NFULL * SBR
    pltpu.sync_copy(x_hbm.at[pl.ds(end - SBR, SBR), pl.ds(ccol, DC)], xv.at[0])
    _process(0, nrows - SBR, SBR - rem)

    GH = G // 2
    STRIP = GH // NS

    def _phase(src_v, glo, out_hbm):
        pltpu.sync_copy(src_v.at[pl.ds(glo, GH)], p_sh.at[sid])
        plsc.subcore_barrier()
        rlo = sid * STRIP
        pltpu.sync_copy(p_sh.at[0, pl.ds(rlo, STRIP)],
                        xv.at[0, pl.ds(0, STRIP)])

        def _racc(j, _):
            pltpu.sync_copy(p_sh.at[j, pl.ds(rlo, STRIP)],
                            xv.at[0, pl.ds(STRIP, STRIP)])

            def _radd(r, _):
                for c in range(NG16):
                    plsc.addupdate(xv.at[0, r, pl.ds(c * 16, 16)],
                                   xv[0, STRIP + r, pl.ds(c * 16, 16)])
                return 0

            lax.fori_loop(0, STRIP, _radd, 0)
            return 0

        lax.fori_loop(1, NS, _racc, 0)
        pltpu.sync_copy(xv.at[0, pl.ds(0, STRIP)],
                        out_hbm.at[pl.ds(glo + rlo, STRIP), pl.ds(ccol, DC)])
        plsc.subcore_barrier()

    _phase(s1v, 0, o1)
    _phase(s1v, GH, o1)
    _phase(s2v, 0, o2)
    _phase(s2v, GH, o2)


def _finalize_body(batch_ref, o1_ref, o2_ref, w_ref, b_ref, s_ref,
                   rb_ref, cnt_ref):
    i = pl.program_id(0)
    nb = pl.num_programs(0)

    @pl.when(i == 0)
    def _init():
        cnt_ref[...] = jnp.zeros_like(cnt_ref)

    bb = batch_ref[0, 0, :]
    onehot = (bb[:, None] == lax.broadcasted_iota(jnp.int32, (BLK, G), 1)
              ).astype(jnp.float32)
    cnt_ref[...] += jnp.sum(onehot, axis=0)[None, :]

    @pl.when(i == nb - 1)
    def _fin():
        s1 = o1_ref[...]
        s2 = o2_ref[...]
        cnt = cnt_ref[0, :][:, None]
        denom = jnp.maximum(cnt, 1.0)
        a = (s1 / denom) * s_ref[...]
        var = (s2 - 2.0 * a * s1 + cnt * a * a) / denom
        r = w_ref[...] * lax.rsqrt(var + 1e-8)
        bp = b_ref[...] - a * r
        rb_ref[...] = jnp.concatenate([r, bp], axis=1)


def _norm_body(x_ref, batch_ref, rb_ref, out_ref):
    bb = batch_ref[0, 0, :]
    onehot = (bb[:, None] == lax.broadcasted_iota(jnp.int32, (BLK, G), 1)
              ).astype(jnp.float32)
    g = lax.dot_general(onehot, rb_ref[...], (((1,), (0,)), ((), ())),
                        preferred_element_type=jnp.float32)
    x = x_ref[...]
    out_ref[...] = x * g[:, :D] + g[:, D:]


@jax.jit
def kernel(node_emb, weight, bias, scale, batch):
    n, d = node_emb.shape
    nb = n // BLK
    batch_i = batch.astype(jnp.int32)

    mesh = plsc.VectorSubcoreMesh(core_axis_name="c", subcore_axis_name="s")
    o1, o2 = pl.kernel(
        _sc_stats_body,
        out_type=(
            jax.ShapeDtypeStruct((G, D), jnp.float32),
            jax.ShapeDtypeStruct((G, D), jnp.float32),
        ),
        mesh=mesh,
        scratch_types=[
            pltpu.VMEM_SHARED((NS, G // 2, DC), jnp.float32),
            pltpu.VMEM((G + 8, DC), jnp.float32),
            pltpu.VMEM((G + 8, DC), jnp.float32),
            pltpu.VMEM((2, SBR, DC), jnp.float32),
            pltpu.VMEM((CHUNK,), jnp.int32),
            pltpu.SMEM((1,), jnp.int32),
            pltpu.SemaphoreType.DMA((2,)),
        ],
    )(node_emb, batch_i)

    w2 = weight.reshape(1, d)
    b2 = bias.reshape(1, d)
    s2 = scale.reshape(1, d)
    batch3 = batch_i.reshape(nb, 1, BLK)
    rb = pl.pallas_call(
        _finalize_body,
        grid=(nb,),
        in_specs=[
            pl.BlockSpec((1, 1, BLK), lambda i: (i, 0, 0)),
            pl.BlockSpec((G, d), lambda i: (0, 0)),
            pl.BlockSpec((G, d), lambda i: (0, 0)),
            pl.BlockSpec((1, d), lambda i: (0, 0)),
            pl.BlockSpec((1, d), lambda i: (0, 0)),
            pl.BlockSpec((1, d), lambda i: (0, 0)),
        ],
        out_specs=pl.BlockSpec((G, 2 * d), lambda i: (0, 0)),
        out_shape=jax.ShapeDtypeStruct((G, 2 * d), jnp.float32),
        scratch_shapes=[pltpu.VMEM((1, G), jnp.float32)],
    )(batch3, o1, o2, w2, b2, s2)
    out = pl.pallas_call(
        _norm_body,
        grid=(nb,),
        in_specs=[
            pl.BlockSpec((BLK, d), lambda i: (i, 0)),
            pl.BlockSpec((1, 1, BLK), lambda i: (i, 0, 0)),
            pl.BlockSpec((G, 2 * d), lambda i: (0, 0)),
        ],
        out_specs=pl.BlockSpec((BLK, d), lambda i: (i, 0)),
        out_shape=jax.ShapeDtypeStruct((n, d), jnp.float32),
    )(node_emb, batch3, rb)
    return out


# speedup vs baseline: 2.2572x; 1.2996x over previous
import jax
import jax.numpy as jnp
from jax import lax
from jax.experimental import pallas as pl
from jax.experimental.pallas import tpu as pltpu
from jax.experimental.pallas import tpu_sc as plsc

N = 50000
D = 256
G = 256
BLK = 2000
SC_LO = 36000
NTC = SC_LO // BLK

NC = 2
NS = 16
DC = D // NC
NG16 = DC // 16
SCN = N - SC_LO
SBR = 80
CHUNK = (SCN // NS + 7) // 8 * 8
NFULL = (SCN // NS - 8) // SBR


def _sc_stats_body(x_hbm, batch_hbm, o1, o2,
                   p_sh, s1v, s2v, xv, idxv, semx):
    cid = lax.axis_index("c")
    sid = lax.axis_index("s")
    ccol = cid * DC
    zeros16 = jnp.zeros((16,), jnp.float32)

    def _zero(r, _):
        for c in range(NG16):
            s1v[r, pl.ds(c * 16, 16)] = zeros16
            s2v[r, pl.ds(c * 16, 16)] = zeros16
        return 0

    lax.fori_loop(0, G, _zero, 0)

    start = SC_LO + sid * SCN // NS // 8 * 8
    end = SC_LO + (sid + 1) * SCN // NS // 8 * 8
    nrows = end - start

    pltpu.sync_copy(batch_hbm.at[pl.ds(start, CHUNK)], idxv)

    def _start_x(k, b):
        return pltpu.async_copy(
            x_hbm.at[pl.ds(start + k * SBR, SBR), pl.ds(ccol, DC)],
            xv.at[b], semx.at[b])

    def _process(b, ioff, lo):
        def _row16(i, _):
            idvec = idxv[pl.ds(ioff + i * 16, 16)]
            g0 = idvec[0]
            g15 = idvec[15]
            a1 = [jnp.zeros((16,), jnp.float32)] * NG16
            a2 = [jnp.zeros((16,), jnp.float32)] * NG16
            for l in range(16):
                r = i * 16 + l
                ma = ((r >= lo) & (idvec[l] == g0)).astype(jnp.float32)
                for c in range(NG16):
                    v = xv[b, r, pl.ds(c * 16, 16)]
                    a1[c] = a1[c] + v * ma
                    a2[c] = a2[c] + (v * v) * ma
            for c in range(NG16):
                plsc.addupdate(s1v.at[g0, pl.ds(c * 16, 16)], a1[c])
                plsc.addupdate(s2v.at[g0, pl.ds(c * 16, 16)], a2[c])

            @pl.when(g15 != g0)
            def _slow():
                idv = idxv[pl.ds(ioff + i * 16, 16)]
                gg0 = idv[0]
                for l in range(16):
                    r = i * 16 + l
                    g = idv[l]
                    mb = ((r >= lo) & (g != gg0)).astype(jnp.float32)
                    for c in range(NG16):
                        v = xv[b, r, pl.ds(c * 16, 16)] * mb
                        plsc.addupdate(s1v.at[g, pl.ds(c * 16, 16)], v)
                        plsc.addupdate(s2v.at[g, pl.ds(c * 16, 16)], v * v)

            return 0

        lax.fori_loop(0, SBR // 16, _row16, 0)

    _start_x(0, 0)
    _start_x(1, 1)

    @pl.loop(0, NFULL // 2 * 2, step=2)
    def _ring(k):
        for b in range(2):
            kk = k + b
            pltpu.make_async_copy(
                x_hbm.at[pl.ds(start + kk * SBR, SBR), pl.ds(ccol, DC)],
                xv.at[b], semx.at[b]).wait()
            _process(b, kk * SBR, 0)

            @pl.when(kk + 2 < NFULL)
            def _next():
                _start_x(kk + 2, b)

    if NFULL % 2:
        kk = NFULL - 1
        pltpu.make_async_copy(
            x_hbm.at[pl.ds(start + kk * SBR, SBR), pl.ds(ccol, DC)],
            xv.at[0], semx.at[0]).wait()
        _process(0, kk * SBR, 0)

    rem = nrows - NFULL * SBR
    pltpu.sync_copy(x_hbm.at[pl.ds(end - SBR, SBR), pl.ds(ccol, DC)], xv.at[0])
    _process(0, nrows - SBR, SBR - rem)

    GH = G // 2
    STRIP = GH // NS

    def _phase(src_v, glo, out_hbm):
        pltpu.sync_copy(src_v.at[pl.ds(glo, GH)], p_sh.at[sid])
        plsc.subcore_barrier()
        rlo = sid * STRIP
        pltpu.sync_copy(p_sh.at[0, pl.ds(rlo, STRIP)],
                        xv.at[0, pl.ds(0, STRIP)])

        def _racc(j, _):
            pltpu.sync_copy(p_sh.at[j, pl.ds(rlo, STRIP)],
                            xv.at[0, pl.ds(STRIP, STRIP)])

            def _radd(r, _):
                for c in range(NG16):
                    plsc.addupdate(xv.at[0, r, pl.ds(c * 16, 16)],
                                   xv[0, STRIP + r, pl.ds(c * 16, 16)])
                return 0

            lax.fori_loop(0, STRIP, _radd, 0)
            return 0

        lax.fori_loop(1, NS, _racc, 0)
        pltpu.sync_copy(xv.at[0, pl.ds(0, STRIP)],
                        out_hbm.at[pl.ds(glo + rlo, STRIP), pl.ds(ccol, DC)])
        plsc.subcore_barrier()

    _phase(s1v, 0, o1)
    _phase(s1v, GH, o1)
    _phase(s2v, 0, o2)
    _phase(s2v, GH, o2)


def _tcstats_body(x_ref, batch_ref, acc_out_ref, acc_ref):
    i = pl.program_id(0)
    nb = pl.num_programs(0)

    @pl.when(i == 0)
    def _init():
        acc_ref[...] = jnp.zeros_like(acc_ref)

    bb = batch_ref[0, 0, :]
    onehot = (bb[:, None] == lax.broadcasted_iota(jnp.int32, (BLK, G), 1)
              ).astype(jnp.float32)
    x = x_ref[...]
    xcat = jnp.concatenate([x, x * x], axis=1)
    acc_ref[...] += lax.dot_general(
        onehot, xcat, (((0,), (0,)), ((), ())),
        preferred_element_type=jnp.float32)

    @pl.when(i == nb - 1)
    def _fin():
        acc_out_ref[...] = acc_ref[...]


def _finalize_body(batch_ref, tc_ref, o1_ref, o2_ref, w_ref, b_ref, s_ref,
                   rb_ref, cnt_ref):
    i = pl.program_id(0)
    nb = pl.num_programs(0)

    @pl.when(i == 0)
    def _init():
        cnt_ref[...] = jnp.zeros_like(cnt_ref)

    bb = batch_ref[0, 0, :]
    onehot = (bb[:, None] == lax.broadcasted_iota(jnp.int32, (BLK, G), 1)
              ).astype(jnp.float32)
    cnt_ref[...] += jnp.sum(onehot, axis=0)[None, :]

    @pl.when(i == nb - 1)
    def _fin():
        s1 = tc_ref[:, :D] + o1_ref[...]
        s2 = tc_ref[:, D:] + o2_ref[...]
        cnt = cnt_ref[0, :][:, None]
        denom = jnp.maximum(cnt, 1.0)
        a = (s1 / denom) * s_ref[...]
        var = (s2 - 2.0 * a * s1 + cnt * a * a) / denom
        r = w_ref[...] * lax.rsqrt(var + 1e-8)
        bp = b_ref[...] - a * r
        rb_ref[...] = jnp.concatenate([r, bp], axis=1)


def _norm_body(x_ref, batch_ref, rb_ref, out_ref):
    bb = batch_ref[0, 0, :]
    onehot = (bb[:, None] == lax.broadcasted_iota(jnp.int32, (BLK, G), 1)
              ).astype(jnp.float32)
    g = lax.dot_general(onehot, rb_ref[...], (((1,), (0,)), ((), ())),
                        preferred_element_type=jnp.float32)
    x = x_ref[...]
    out_ref[...] = x * g[:, :D] + g[:, D:]


@jax.jit
def kernel(node_emb, weight, bias, scale, batch):
    n, d = node_emb.shape
    nb = n // BLK
    batch_i = batch.astype(jnp.int32)
    batch3 = batch_i.reshape(nb, 1, BLK)

    mesh = plsc.VectorSubcoreMesh(core_axis_name="c", subcore_axis_name="s")
    o1, o2 = pl.kernel(
        _sc_stats_body,
        out_type=(
            jax.ShapeDtypeStruct((G, D), jnp.float32),
            jax.ShapeDtypeStruct((G, D), jnp.float32),
        ),
        mesh=mesh,
        scratch_types=[
            pltpu.VMEM_SHARED((NS, G // 2, DC), jnp.float32),
            pltpu.VMEM((G, DC), jnp.float32),
            pltpu.VMEM((G, DC), jnp.float32),
            pltpu.VMEM((2, SBR, DC), jnp.float32),
            pltpu.VMEM((CHUNK,), jnp.int32),
            pltpu.SemaphoreType.DMA((2,)),
        ],
    )(node_emb, batch_i)

    tc_acc = pl.pallas_call(
        _tcstats_body,
        grid=(NTC,),
        in_specs=[
            pl.BlockSpec((BLK, d), lambda i: (i, 0)),
            pl.BlockSpec((1, 1, BLK), lambda i: (i, 0, 0)),
        ],
        out_specs=pl.BlockSpec((G, 2 * d), lambda i: (0, 0)),
        out_shape=jax.ShapeDtypeStruct((G, 2 * d), jnp.float32),
        scratch_shapes=[pltpu.VMEM((G, 2 * d), jnp.float32)],
    )(node_emb, batch3)

    w2 = weight.reshape(1, d)
    b2 = bias.reshape(1, d)
    s2 = scale.reshape(1, d)
    rb = pl.pallas_call(
        _finalize_body,
        grid=(nb,),
        in_specs=[
            pl.BlockSpec((1, 1, BLK), lambda i: (i, 0, 0)),
            pl.BlockSpec((G, 2 * d), lambda i: (0, 0)),
            pl.BlockSpec((G, d), lambda i: (0, 0)),
            pl.BlockSpec((G, d), lambda i: (0, 0)),
            pl.BlockSpec((1, d), lambda i: (0, 0)),
            pl.BlockSpec((1, d), lambda i: (0, 0)),
            pl.BlockSpec((1, d), lambda i: (0, 0)),
        ],
        out_specs=pl.BlockSpec((G, 2 * d), lambda i: (0, 0)),
        out_shape=jax.ShapeDtypeStruct((G, 2 * d), jnp.float32),
        scratch_shapes=[pltpu.VMEM((1, G), jnp.float32)],
    )(batch3, tc_acc, o1, o2, w2, b2, s2)

    out = pl.pallas_call(
        _norm_body,
        grid=(nb,),
        in_specs=[
            pl.BlockSpec((BLK, d), lambda i: (i, 0)),
            pl.BlockSpec((1, 1, BLK), lambda i: (i, 0, 0)),
            pl.BlockSpec((G, 2 * d), lambda i: (0, 0)),
        ],
        out_specs=pl.BlockSpec((BLK, d), lambda i: (i, 0)),
        out_shape=jax.ShapeDtypeStruct((n, d), jnp.float32),
    )(node_emb, batch3, rb)
    return out


# speedup vs baseline: 2.4260x; 1.0748x over previous
import jax
import jax.numpy as jnp
from jax import lax
from jax.experimental import pallas as pl
from jax.experimental.pallas import tpu as pltpu
from jax.experimental.pallas import tpu_sc as plsc

N = 50000
D = 256
G = 256
BLK = 2000
SC_LO = 36000
NTC = SC_LO // BLK

NC = 2
NS = 16
DC = D // NC
NG16 = DC // 16
SCN = N - SC_LO
SBR = 80
CHUNK = (SCN // NS + 7) // 8 * 8
NFULL = (SCN // NS - 8) // SBR


def _sc_stats_body(x_hbm, batch_hbm, o1, o2,
                   p_sh, s1v, s2v, xv, idxv, semx):
    cid = lax.axis_index("c")
    sid = lax.axis_index("s")
    ccol = cid * DC
    zeros16 = jnp.zeros((16,), jnp.float32)

    def _zero(r, _):
        for c in range(NG16):
            s1v[r, pl.ds(c * 16, 16)] = zeros16
            s2v[r, pl.ds(c * 16, 16)] = zeros16
        return 0

    lax.fori_loop(0, G, _zero, 0)

    start = SC_LO + sid * SCN // NS // 8 * 8
    end = SC_LO + (sid + 1) * SCN // NS // 8 * 8
    nrows = end - start

    pltpu.sync_copy(batch_hbm.at[pl.ds(start, CHUNK)], idxv)

    def _start_x(k, b):
        return pltpu.async_copy(
            x_hbm.at[pl.ds(start + k * SBR, SBR), pl.ds(ccol, DC)],
            xv.at[b], semx.at[b])

    def _process(b, ioff, lo):
        def _row16(i, _):
            idvec = idxv[pl.ds(ioff + i * 16, 16)]
            g0 = idvec[0]
            g15 = idvec[15]
            a1 = [jnp.zeros((16,), jnp.float32)] * NG16
            a2 = [jnp.zeros((16,), jnp.float32)] * NG16
            for l in range(16):
                r = i * 16 + l
                ma = ((r >= lo) & (idvec[l] == g0)).astype(jnp.float32)
                for c in range(NG16):
                    v = xv[b, r, pl.ds(c * 16, 16)]
                    a1[c] = a1[c] + v * ma
                    a2[c] = a2[c] + (v * v) * ma
            for c in range(NG16):
                plsc.addupdate(s1v.at[g0, pl.ds(c * 16, 16)], a1[c])
                plsc.addupdate(s2v.at[g0, pl.ds(c * 16, 16)], a2[c])

            @pl.when(g15 != g0)
            def _slow():
                idv = idxv[pl.ds(ioff + i * 16, 16)]
                gg0 = idv[0]
                for l in range(16):
                    r = i * 16 + l
                    g = idv[l]
                    mb = ((r >= lo) & (g != gg0)).astype(jnp.float32)
                    for c in range(NG16):
                        v = xv[b, r, pl.ds(c * 16, 16)] * mb
                        plsc.addupdate(s1v.at[g, pl.ds(c * 16, 16)], v)
                        plsc.addupdate(s2v.at[g, pl.ds(c * 16, 16)], v * v)

            return 0

        lax.fori_loop(0, SBR // 16, _row16, 0)

    _start_x(0, 0)
    _start_x(1, 1)

    @pl.loop(0, NFULL // 2 * 2, step=2)
    def _ring(k):
        for b in range(2):
            kk = k + b
            pltpu.make_async_copy(
                x_hbm.at[pl.ds(start + kk * SBR, SBR), pl.ds(ccol, DC)],
                xv.at[b], semx.at[b]).wait()
            _process(b, kk * SBR, 0)

            @pl.when(kk + 2 < NFULL)
            def _next():
                _start_x(kk + 2, b)

    if NFULL % 2:
        kk = NFULL - 1
        pltpu.make_async_copy(
            x_hbm.at[pl.ds(start + kk * SBR, SBR), pl.ds(ccol, DC)],
            xv.at[0], semx.at[0]).wait()
        _process(0, kk * SBR, 0)

    rem = nrows - NFULL * SBR
    pltpu.sync_copy(x_hbm.at[pl.ds(end - SBR, SBR), pl.ds(ccol, DC)], xv.at[0])
    _process(0, nrows - SBR, SBR - rem)

    GH = G // 2
    STRIP = GH // NS

    def _phase(src_v, glo, out_hbm):
        pltpu.sync_copy(src_v.at[pl.ds(glo, GH)], p_sh.at[sid])
        plsc.subcore_barrier()
        rlo = sid * STRIP
        pltpu.sync_copy(p_sh.at[0, pl.ds(rlo, STRIP)],
                        xv.at[0, pl.ds(0, STRIP)])

        def _racc(j, _):
            pltpu.sync_copy(p_sh.at[j, pl.ds(rlo, STRIP)],
                            xv.at[0, pl.ds(STRIP, STRIP)])

            def _radd(r, _):
                for c in range(NG16):
                    plsc.addupdate(xv.at[0, r, pl.ds(c * 16, 16)],
                                   xv[0, STRIP + r, pl.ds(c * 16, 16)])
                return 0

            lax.fori_loop(0, STRIP, _radd, 0)
            return 0

        lax.fori_loop(1, NS, _racc, 0)
        pltpu.sync_copy(xv.at[0, pl.ds(0, STRIP)],
                        out_hbm.at[pl.ds(glo + rlo, STRIP), pl.ds(ccol, DC)])
        plsc.subcore_barrier()

    _phase(s1v, 0, o1)
    _phase(s1v, GH, o1)
    _phase(s2v, 0, o2)
    _phase(s2v, GH, o2)


def _tcstats_body(x_ref, batch_ref, acc_out_ref, acc_ref):
    i = pl.program_id(0)
    nb = pl.num_programs(0)

    @pl.when(i == 0)
    def _init():
        acc_ref[...] = jnp.zeros_like(acc_ref)

    bb = batch_ref[0, 0, :]
    onehot = (bb[:, None] == lax.broadcasted_iota(jnp.int32, (BLK, G), 1)
              ).astype(jnp.bfloat16)
    x = x_ref[...].astype(jnp.bfloat16)
    ones = jnp.ones((BLK, 8), jnp.bfloat16)
    xcat = jnp.concatenate([x, x * x, ones], axis=1)
    acc_ref[...] += lax.dot_general(
        onehot, xcat, (((0,), (0,)), ((), ())),
        preferred_element_type=jnp.float32)

    @pl.when(i == nb - 1)
    def _fin():
        acc_out_ref[...] = acc_ref[...]


def _finalize_body(batch_ref, tc_ref, o1_ref, o2_ref, w_ref, b_ref, s_ref,
                   rb_ref, cnt_ref):
    i = pl.program_id(0)
    nb = pl.num_programs(0)

    @pl.when(i == 0)
    def _init():
        cnt_ref[...] = jnp.zeros_like(cnt_ref)

    bb = batch_ref[0, 0, :]
    onehot = (bb[:, None] == lax.broadcasted_iota(jnp.int32, (BLK, G), 1)
              ).astype(jnp.float32)
    cnt_ref[...] += jnp.sum(onehot, axis=0)[None, :]

    @pl.when(i == nb - 1)
    def _fin():
        s1 = tc_ref[:, :D] + o1_ref[...]
        s2 = tc_ref[:, D:2 * D] + o2_ref[...]
        cnt = cnt_ref[0, :][:, None] + tc_ref[:, 2 * D:2 * D + 1]
        denom = jnp.maximum(cnt, 1.0)
        a = (s1 / denom) * s_ref[...]
        var = (s2 - 2.0 * a * s1 + cnt * a * a) / denom
        r = w_ref[...] * lax.rsqrt(var + 1e-8)
        bp = b_ref[...] - a * r
        rb_ref[...] = jnp.concatenate([r, bp], axis=1)


def _norm_body(x_ref, batch_ref, rb_ref, out_ref):
    bb = batch_ref[0, 0, :]
    onehot = (bb[:, None] == lax.broadcasted_iota(jnp.int32, (BLK, G), 1)
              ).astype(jnp.bfloat16)
    g = lax.dot_general(onehot, rb_ref[...].astype(jnp.bfloat16),
                        (((1,), (0,)), ((), ())),
                        preferred_element_type=jnp.float32)
    x = x_ref[...]
    out_ref[...] = x * g[:, :D] + g[:, D:]


@jax.jit
def kernel(node_emb, weight, bias, scale, batch):
    n, d = node_emb.shape
    nb = n // BLK
    batch_i = batch.astype(jnp.int32)
    batch3 = batch_i.reshape(nb, 1, BLK)

    mesh = plsc.VectorSubcoreMesh(core_axis_name="c", subcore_axis_name="s")
    o1, o2 = pl.kernel(
        _sc_stats_body,
        out_type=(
            jax.ShapeDtypeStruct((G, D), jnp.float32),
            jax.ShapeDtypeStruct((G, D), jnp.float32),
        ),
        mesh=mesh,
        scratch_types=[
            pltpu.VMEM_SHARED((NS, G // 2, DC), jnp.float32),
            pltpu.VMEM((G, DC), jnp.float32),
            pltpu.VMEM((G, DC), jnp.float32),
            pltpu.VMEM((2, SBR, DC), jnp.float32),
            pltpu.VMEM((CHUNK,), jnp.int32),
            pltpu.SemaphoreType.DMA((2,)),
        ],
    )(node_emb, batch_i)

    tc_acc = pl.pallas_call(
        _tcstats_body,
        grid=(NTC,),
        in_specs=[
            pl.BlockSpec((BLK, d), lambda i: (i, 0)),
            pl.BlockSpec((1, 1, BLK), lambda i: (i, 0, 0)),
        ],
        out_specs=pl.BlockSpec((G, 2 * d + 8), lambda i: (0, 0)),
        out_shape=jax.ShapeDtypeStruct((G, 2 * d + 8), jnp.float32),
        scratch_shapes=[pltpu.VMEM((G, 2 * d + 8), jnp.float32)],
    )(node_emb, batch3)

    w2 = weight.reshape(1, d)
    b2 = bias.reshape(1, d)
    s2 = scale.reshape(1, d)
    rb = pl.pallas_call(
        _finalize_body,
        grid=(nb - NTC,),
        in_specs=[
            pl.BlockSpec((1, 1, BLK), lambda i: (NTC + i, 0, 0)),
            pl.BlockSpec((G, 2 * d + 8), lambda i: (0, 0)),
            pl.BlockSpec((G, d), lambda i: (0, 0)),
            pl.BlockSpec((G, d), lambda i: (0, 0)),
            pl.BlockSpec((1, d), lambda i: (0, 0)),
            pl.BlockSpec((1, d), lambda i: (0, 0)),
            pl.BlockSpec((1, d), lambda i: (0, 0)),
        ],
        out_specs=pl.BlockSpec((G, 2 * d), lambda i: (0, 0)),
        out_shape=jax.ShapeDtypeStruct((G, 2 * d), jnp.float32),
        scratch_shapes=[pltpu.VMEM((1, G), jnp.float32)],
    )(batch3, tc_acc, o1, o2, w2, b2, s2)

    out = pl.pallas_call(
        _norm_body,
        grid=(nb,),
        in_specs=[
            pl.BlockSpec((BLK, d), lambda i: (i, 0)),
            pl.BlockSpec((1, 1, BLK), lambda i: (i, 0, 0)),
            pl.BlockSpec((G, 2 * d), lambda i: (0, 0)),
        ],
        out_specs=pl.BlockSpec((BLK, d), lambda i: (i, 0)),
        out_shape=jax.ShapeDtypeStruct((n, d), jnp.float32),
    )(node_emb, batch3, rb)
    return out


# speedup vs baseline: 2.6581x; 1.0957x over previous
import jax
import jax.numpy as jnp
from jax import lax
from jax.experimental import pallas as pl
from jax.experimental.pallas import tpu as pltpu
from jax.experimental.pallas import tpu_sc as plsc

N = 50000
D = 256
G = 256
BLK = 2000
SC_LO = 44000
NTC = SC_LO // BLK

NC = 2
NS = 16
DC = D // NC
NG16 = DC // 16
SCN = N - SC_LO
SBR = 80
CHUNK = (SCN // NS + 7) // 8 * 8
NFULL = (SCN // NS - 8) // SBR


def _sc_stats_body(x_hbm, batch_hbm, o1, o2,
                   p_sh, s1v, s2v, xv, idxv, semx):
    cid = lax.axis_index("c")
    sid = lax.axis_index("s")
    ccol = cid * DC
    zeros16 = jnp.zeros((16,), jnp.float32)

    def _zero(r, _):
        for c in range(NG16):
            s1v[r, pl.ds(c * 16, 16)] = zeros16
            s2v[r, pl.ds(c * 16, 16)] = zeros16
        return 0

    lax.fori_loop(0, G, _zero, 0)

    start = SC_LO + sid * SCN // NS // 8 * 8
    end = SC_LO + (sid + 1) * SCN // NS // 8 * 8
    nrows = end - start

    pltpu.sync_copy(batch_hbm.at[pl.ds(start, CHUNK)], idxv)

    def _start_x(k, b):
        return pltpu.async_copy(
            x_hbm.at[pl.ds(start + k * SBR, SBR), pl.ds(ccol, DC)],
            xv.at[b], semx.at[b])

    def _process(b, ioff, lo):
        def _row16(i, _):
            idvec = idxv[pl.ds(ioff + i * 16, 16)]
            g0 = idvec[0]
            g15 = idvec[15]
            a1 = [jnp.zeros((16,), jnp.float32)] * NG16
            a2 = [jnp.zeros((16,), jnp.float32)] * NG16
            for l in range(16):
                r = i * 16 + l
                ma = ((r >= lo) & (idvec[l] == g0)).astype(jnp.float32)
                for c in range(NG16):
                    v = xv[b, r, pl.ds(c * 16, 16)]
                    a1[c] = a1[c] + v * ma
                    a2[c] = a2[c] + (v * v) * ma
            for c in range(NG16):
                plsc.addupdate(s1v.at[g0, pl.ds(c * 16, 16)], a1[c])
                plsc.addupdate(s2v.at[g0, pl.ds(c * 16, 16)], a2[c])

            @pl.when(g15 != g0)
            def _slow():
                idv = idxv[pl.ds(ioff + i * 16, 16)]
                gg0 = idv[0]
                for l in range(16):
                    r = i * 16 + l
                    g = idv[l]
                    mb = ((r >= lo) & (g != gg0)).astype(jnp.float32)
                    for c in range(NG16):
                        v = xv[b, r, pl.ds(c * 16, 16)] * mb
                        plsc.addupdate(s1v.at[g, pl.ds(c * 16, 16)], v)
                        plsc.addupdate(s2v.at[g, pl.ds(c * 16, 16)], v * v)

            return 0

        lax.fori_loop(0, SBR // 16, _row16, 0)

    _start_x(0, 0)
    _start_x(1, 1)

    @pl.loop(0, NFULL // 2 * 2, step=2)
    def _ring(k):
        for b in range(2):
            kk = k + b
            pltpu.make_async_copy(
                x_hbm.at[pl.ds(start + kk * SBR, SBR), pl.ds(ccol, DC)],
                xv.at[b], semx.at[b]).wait()
            _process(b, kk * SBR, 0)

            @pl.when(kk + 2 < NFULL)
            def _next():
                _start_x(kk + 2, b)

    if NFULL % 2:
        kk = NFULL - 1
        pltpu.make_async_copy(
            x_hbm.at[pl.ds(start + kk * SBR, SBR), pl.ds(ccol, DC)],
            xv.at[0], semx.at[0]).wait()
        _process(0, kk * SBR, 0)

    rem = nrows - NFULL * SBR
    pltpu.sync_copy(x_hbm.at[pl.ds(end - SBR, SBR), pl.ds(ccol, DC)], xv.at[0])
    _process(0, nrows - SBR, SBR - rem)

    GH = G // 2
    STRIP = GH // NS

    def _phase(src_v, glo, out_hbm):
        pltpu.sync_copy(src_v.at[pl.ds(glo, GH)], p_sh.at[sid])
        plsc.subcore_barrier()
        rlo = sid * STRIP
        pltpu.sync_copy(p_sh.at[0, pl.ds(rlo, STRIP)],
                        xv.at[0, pl.ds(0, STRIP)])

        def _racc(j, _):
            pltpu.sync_copy(p_sh.at[j, pl.ds(rlo, STRIP)],
                            xv.at[0, pl.ds(STRIP, STRIP)])

            def _radd(r, _):
                for c in range(NG16):
                    plsc.addupdate(xv.at[0, r, pl.ds(c * 16, 16)],
                                   xv[0, STRIP + r, pl.ds(c * 16, 16)])
                return 0

            lax.fori_loop(0, STRIP, _radd, 0)
            return 0

        lax.fori_loop(1, NS, _racc, 0)
        pltpu.sync_copy(xv.at[0, pl.ds(0, STRIP)],
                        out_hbm.at[pl.ds(glo + rlo, STRIP), pl.ds(ccol, DC)])
        plsc.subcore_barrier()

    _phase(s1v, 0, o1)
    _phase(s1v, GH, o1)
    _phase(s2v, 0, o2)
    _phase(s2v, GH, o2)


def _tcstats_body(x_ref, batch_ref, acc_out_ref, acc_ref):
    i = pl.program_id(0)
    nb = pl.num_programs(0)

    @pl.when(i == 0)
    def _init():
        acc_ref[...] = jnp.zeros_like(acc_ref)

    bb = batch_ref[0, 0, :]
    onehot = (bb[:, None] == lax.broadcasted_iota(jnp.int32, (BLK, G), 1)
              ).astype(jnp.bfloat16)
    x = x_ref[...].astype(jnp.bfloat16)
    ones = jnp.ones((BLK, 8), jnp.bfloat16)
    xcat = jnp.concatenate([x, x * x, ones], axis=1)
    acc_ref[...] += lax.dot_general(
        onehot, xcat, (((0,), (0,)), ((), ())),
        preferred_element_type=jnp.float32)

    @pl.when(i == nb - 1)
    def _fin():
        acc_out_ref[...] = acc_ref[...]


def _finalize_body(batch_ref, tc_ref, o1_ref, o2_ref, w_ref, b_ref, s_ref,
                   rb_ref, cnt_ref):
    i = pl.program_id(0)
    nb = pl.num_programs(0)

    @pl.when(i == 0)
    def _init():
        cnt_ref[...] = jnp.zeros_like(cnt_ref)

    bb = batch_ref[0, 0, :]
    onehot = (bb[:, None] == lax.broadcasted_iota(jnp.int32, (BLK, G), 1)
              ).astype(jnp.float32)
    cnt_ref[...] += jnp.sum(onehot, axis=0)[None, :]

    @pl.when(i == nb - 1)
    def _fin():
        s1 = tc_ref[:, :D] + o1_ref[...]
        s2 = tc_ref[:, D:2 * D] + o2_ref[...]
        cnt = cnt_ref[0, :][:, None] + tc_ref[:, 2 * D:2 * D + 1]
        denom = jnp.maximum(cnt, 1.0)
        a = (s1 / denom) * s_ref[...]
        var = (s2 - 2.0 * a * s1 + cnt * a * a) / denom
        r = w_ref[...] * lax.rsqrt(var + 1e-8)
        bp = b_ref[...] - a * r
        rb_ref[...] = jnp.concatenate([r, bp], axis=1)


def _norm_body(x_ref, batch_ref, rb_ref, out_ref):
    bb = batch_ref[0, 0, :]
    onehot = (bb[:, None] == lax.broadcasted_iota(jnp.int32, (BLK, G), 1)
              ).astype(jnp.bfloat16)
    g = lax.dot_general(onehot, rb_ref[...].astype(jnp.bfloat16),
                        (((1,), (0,)), ((), ())),
                        preferred_element_type=jnp.float32)
    x = x_ref[...]
    out_ref[...] = x * g[:, :D] + g[:, D:]


@jax.jit
def kernel(node_emb, weight, bias, scale, batch):
    n, d = node_emb.shape
    nb = n // BLK
    batch_i = batch.astype(jnp.int32)
    batch3 = batch_i.reshape(nb, 1, BLK)

    mesh = plsc.VectorSubcoreMesh(core_axis_name="c", subcore_axis_name="s")
    o1, o2 = pl.kernel(
        _sc_stats_body,
        out_type=(
            jax.ShapeDtypeStruct((G, D), jnp.float32),
            jax.ShapeDtypeStruct((G, D), jnp.float32),
        ),
        mesh=mesh,
        scratch_types=[
            pltpu.VMEM_SHARED((NS, G // 2, DC), jnp.float32),
            pltpu.VMEM((G, DC), jnp.float32),
            pltpu.VMEM((G, DC), jnp.float32),
            pltpu.VMEM((2, SBR, DC), jnp.float32),
            pltpu.VMEM((CHUNK,), jnp.int32),
            pltpu.SemaphoreType.DMA((2,)),
        ],
    )(node_emb, batch_i)

    tc_acc = pl.pallas_call(
        _tcstats_body,
        grid=(NTC,),
        in_specs=[
            pl.BlockSpec((BLK, d), lambda i: (i, 0)),
            pl.BlockSpec((1, 1, BLK), lambda i: (i, 0, 0)),
        ],
        out_specs=pl.BlockSpec((G, 2 * d + 8), lambda i: (0, 0)),
        out_shape=jax.ShapeDtypeStruct((G, 2 * d + 8), jnp.float32),
        scratch_shapes=[pltpu.VMEM((G, 2 * d + 8), jnp.float32)],
    )(node_emb, batch3)

    w2 = weight.reshape(1, d)
    b2 = bias.reshape(1, d)
    s2 = scale.reshape(1, d)
    rb = pl.pallas_call(
        _finalize_body,
        grid=(nb - NTC,),
        in_specs=[
            pl.BlockSpec((1, 1, BLK), lambda i: (NTC + i, 0, 0)),
            pl.BlockSpec((G, 2 * d + 8), lambda i: (0, 0)),
            pl.BlockSpec((G, d), lambda i: (0, 0)),
            pl.BlockSpec((G, d), lambda i: (0, 0)),
            pl.BlockSpec((1, d), lambda i: (0, 0)),
            pl.BlockSpec((1, d), lambda i: (0, 0)),
            pl.BlockSpec((1, d), lambda i: (0, 0)),
        ],
        out_specs=pl.BlockSpec((G, 2 * d), lambda i: (0, 0)),
        out_shape=jax.ShapeDtypeStruct((G, 2 * d), jnp.float32),
        scratch_shapes=[pltpu.VMEM((1, G), jnp.float32)],
    )(batch3, tc_acc, o1, o2, w2, b2, s2)

    out = pl.pallas_call(
        _norm_body,
        grid=(nb,),
        in_specs=[
            pl.BlockSpec((BLK, d), lambda i: (i, 0)),
            pl.BlockSpec((1, 1, BLK), lambda i: (i, 0, 0)),
            pl.BlockSpec((G, 2 * d), lambda i: (0, 0)),
        ],
        out_specs=pl.BlockSpec((BLK, d), lambda i: (i, 0)),
        out_shape=jax.ShapeDtypeStruct((n, d), jnp.float32),
    )(node_emb, batch3, rb)
    return out


# speedup vs baseline: 2.7718x; 1.0428x over previous
import jax
import jax.numpy as jnp
from jax import lax
from jax.experimental import pallas as pl
from jax.experimental.pallas import tpu as pltpu
from jax.experimental.pallas import tpu_sc as plsc

N = 50000
D = 256
G = 256
BLK = 2000
SC_LO = 48000
NTC = SC_LO // BLK

NC = 2
NS = 16
DC = D // NC
NG16 = DC // 16
SCN = N - SC_LO
SBR = 80
CHUNK = (SCN // NS + 7) // 8 * 8
NFULL = (SCN // NS - 8) // SBR


def _sc_stats_body(x_hbm, batch_hbm, o1, o2,
                   p_sh, s1v, s2v, xv, idxv, semx):
    cid = lax.axis_index("c")
    sid = lax.axis_index("s")
    ccol = cid * DC
    zeros16 = jnp.zeros((16,), jnp.float32)

    def _zero(r, _):
        for c in range(NG16):
            s1v[r, pl.ds(c * 16, 16)] = zeros16
            s2v[r, pl.ds(c * 16, 16)] = zeros16
        return 0

    lax.fori_loop(0, G, _zero, 0)

    start = SC_LO + sid * SCN // NS // 8 * 8
    end = SC_LO + (sid + 1) * SCN // NS // 8 * 8
    nrows = end - start

    pltpu.sync_copy(batch_hbm.at[pl.ds(start, CHUNK)], idxv)

    def _start_x(k, b):
        return pltpu.async_copy(
            x_hbm.at[pl.ds(start + k * SBR, SBR), pl.ds(ccol, DC)],
            xv.at[b], semx.at[b])

    def _process(b, ioff, lo):
        def _row16(i, _):
            idvec = idxv[pl.ds(ioff + i * 16, 16)]
            g0 = idvec[0]
            g15 = idvec[15]
            a1 = [jnp.zeros((16,), jnp.float32)] * NG16
            a2 = [jnp.zeros((16,), jnp.float32)] * NG16
            for l in range(16):
                r = i * 16 + l
                ma = ((r >= lo) & (idvec[l] == g0)).astype(jnp.float32)
                for c in range(NG16):
                    v = xv[b, r, pl.ds(c * 16, 16)]
                    a1[c] = a1[c] + v * ma
                    a2[c] = a2[c] + (v * v) * ma
            for c in range(NG16):
                plsc.addupdate(s1v.at[g0, pl.ds(c * 16, 16)], a1[c])
                plsc.addupdate(s2v.at[g0, pl.ds(c * 16, 16)], a2[c])

            @pl.when(g15 != g0)
            def _slow():
                idv = idxv[pl.ds(ioff + i * 16, 16)]
                gg0 = idv[0]
                for l in range(16):
                    r = i * 16 + l
                    g = idv[l]
                    mb = ((r >= lo) & (g != gg0)).astype(jnp.float32)
                    for c in range(NG16):
                        v = xv[b, r, pl.ds(c * 16, 16)] * mb
                        plsc.addupdate(s1v.at[g, pl.ds(c * 16, 16)], v)
                        plsc.addupdate(s2v.at[g, pl.ds(c * 16, 16)], v * v)

            return 0

        lax.fori_loop(0, SBR // 16, _row16, 0)

    _start_x(0, 0)
    _start_x(1, 1)

    @pl.loop(0, NFULL // 2 * 2, step=2)
    def _ring(k):
        for b in range(2):
            kk = k + b
            pltpu.make_async_copy(
                x_hbm.at[pl.ds(start + kk * SBR, SBR), pl.ds(ccol, DC)],
                xv.at[b], semx.at[b]).wait()
            _process(b, kk * SBR, 0)

            @pl.when(kk + 2 < NFULL)
            def _next():
                _start_x(kk + 2, b)

    if NFULL % 2:
        kk = NFULL - 1
        pltpu.make_async_copy(
            x_hbm.at[pl.ds(start + kk * SBR, SBR), pl.ds(ccol, DC)],
            xv.at[0], semx.at[0]).wait()
        _process(0, kk * SBR, 0)

    rem = nrows - NFULL * SBR
    pltpu.sync_copy(x_hbm.at[pl.ds(end - SBR, SBR), pl.ds(ccol, DC)], xv.at[0])
    _process(0, nrows - SBR, SBR - rem)

    GH = G // 2
    STRIP = GH // NS

    def _phase(src_v, glo, out_hbm):
        pltpu.sync_copy(src_v.at[pl.ds(glo, GH)], p_sh.at[sid])
        plsc.subcore_barrier()
        rlo = sid * STRIP
        pltpu.sync_copy(p_sh.at[0, pl.ds(rlo, STRIP)],
                        xv.at[0, pl.ds(0, STRIP)])

        def _racc(j, _):
            pltpu.sync_copy(p_sh.at[j, pl.ds(rlo, STRIP)],
                            xv.at[0, pl.ds(STRIP, STRIP)])

            def _radd(r, _):
                for c in range(NG16):
                    plsc.addupdate(xv.at[0, r, pl.ds(c * 16, 16)],
                                   xv[0, STRIP + r, pl.ds(c * 16, 16)])
                return 0

            lax.fori_loop(0, STRIP, _radd, 0)
            return 0

        lax.fori_loop(1, NS, _racc, 0)
        pltpu.sync_copy(xv.at[0, pl.ds(0, STRIP)],
                        out_hbm.at[pl.ds(glo + rlo, STRIP), pl.ds(ccol, DC)])
        plsc.subcore_barrier()

    _phase(s1v, 0, o1)
    _phase(s1v, GH, o1)
    _phase(s2v, 0, o2)
    _phase(s2v, GH, o2)


def _tcstats_body(x_ref, batch_ref, acc_out_ref, acc_ref):
    i = pl.program_id(0)
    nb = pl.num_programs(0)

    @pl.when(i == 0)
    def _init():
        acc_ref[...] = jnp.zeros_like(acc_ref)

    bb = batch_ref[0, 0, :]
    onehot = (bb[:, None] == lax.broadcasted_iota(jnp.int32, (BLK, G), 1)
              ).astype(jnp.bfloat16)
    x = x_ref[...].astype(jnp.bfloat16)
    ones = jnp.ones((BLK, 8), jnp.bfloat16)
    xcat = jnp.concatenate([x, x * x, ones], axis=1)
    acc_ref[...] += lax.dot_general(
        onehot, xcat, (((0,), (0,)), ((), ())),
        preferred_element_type=jnp.float32)

    @pl.when(i == nb - 1)
    def _fin():
        acc_out_ref[...] = acc_ref[...]


def _finalize_body(batch_ref, tc_ref, o1_ref, o2_ref, w_ref, b_ref, s_ref,
                   rb_ref, cnt_ref):
    i = pl.program_id(0)
    nb = pl.num_programs(0)

    @pl.when(i == 0)
    def _init():
        cnt_ref[...] = jnp.zeros_like(cnt_ref)

    bb = batch_ref[0, 0, :]
    onehot = (bb[:, None] == lax.broadcasted_iota(jnp.int32, (BLK, G), 1)
              ).astype(jnp.float32)
    cnt_ref[...] += jnp.sum(onehot, axis=0)[None, :]

    @pl.when(i == nb - 1)
    def _fin():
        s1 = tc_ref[:, :D] + o1_ref[...]
        s2 = tc_ref[:, D:2 * D] + o2_ref[...]
        cnt = cnt_ref[0, :][:, None] + tc_ref[:, 2 * D:2 * D + 1]
        denom = jnp.maximum(cnt, 1.0)
        a = (s1 / denom) * s_ref[...]
        var = (s2 - 2.0 * a * s1 + cnt * a * a) / denom
        r = w_ref[...] * lax.rsqrt(var + 1e-8)
        bp = b_ref[...] - a * r
        rb_ref[...] = jnp.concatenate([r, bp], axis=1)


def _norm_body(x_ref, batch_ref, rb_ref, out_ref):
    bb = batch_ref[0, 0, :]
    onehot = (bb[:, None] == lax.broadcasted_iota(jnp.int32, (BLK, G), 1)
              ).astype(jnp.bfloat16)
    g = lax.dot_general(onehot, rb_ref[...].astype(jnp.bfloat16),
                        (((1,), (0,)), ((), ())),
                        preferred_element_type=jnp.float32)
    x = x_ref[...]
    out_ref[...] = x * g[:, :D] + g[:, D:]


@jax.jit
def kernel(node_emb, weight, bias, scale, batch):
    n, d = node_emb.shape
    nb = n // BLK
    batch_i = batch.astype(jnp.int32)
    batch3 = batch_i.reshape(nb, 1, BLK)

    mesh = plsc.VectorSubcoreMesh(core_axis_name="c", subcore_axis_name="s")
    o1, o2 = pl.kernel(
        _sc_stats_body,
        out_type=(
            jax.ShapeDtypeStruct((G, D), jnp.float32),
            jax.ShapeDtypeStruct((G, D), jnp.float32),
        ),
        mesh=mesh,
        scratch_types=[
            pltpu.VMEM_SHARED((NS, G // 2, DC), jnp.float32),
            pltpu.VMEM((G, DC), jnp.float32),
            pltpu.VMEM((G, DC), jnp.float32),
            pltpu.VMEM((2, SBR, DC), jnp.float32),
            pltpu.VMEM((CHUNK,), jnp.int32),
            pltpu.SemaphoreType.DMA((2,)),
        ],
    )(node_emb, batch_i)

    tc_acc = pl.pallas_call(
        _tcstats_body,
        grid=(NTC,),
        in_specs=[
            pl.BlockSpec((BLK, d), lambda i: (i, 0)),
            pl.BlockSpec((1, 1, BLK), lambda i: (i, 0, 0)),
        ],
        out_specs=pl.BlockSpec((G, 2 * d + 8), lambda i: (0, 0)),
        out_shape=jax.ShapeDtypeStruct((G, 2 * d + 8), jnp.float32),
        scratch_shapes=[pltpu.VMEM((G, 2 * d + 8), jnp.float32)],
    )(node_emb, batch3)

    w2 = weight.reshape(1, d)
    b2 = bias.reshape(1, d)
    s2 = scale.reshape(1, d)
    rb = pl.pallas_call(
        _finalize_body,
        grid=(nb - NTC,),
        in_specs=[
            pl.BlockSpec((1, 1, BLK), lambda i: (NTC + i, 0, 0)),
            pl.BlockSpec((G, 2 * d + 8), lambda i: (0, 0)),
            pl.BlockSpec((G, d), lambda i: (0, 0)),
            pl.BlockSpec((G, d), lambda i: (0, 0)),
            pl.BlockSpec((1, d), lambda i: (0, 0)),
            pl.BlockSpec((1, d), lambda i: (0, 0)),
            pl.BlockSpec((1, d), lambda i: (0, 0)),
        ],
        out_specs=pl.BlockSpec((G, 2 * d), lambda i: (0, 0)),
        out_shape=jax.ShapeDtypeStruct((G, 2 * d), jnp.float32),
        scratch_shapes=[pltpu.VMEM((1, G), jnp.float32)],
    )(batch3, tc_acc, o1, o2, w2, b2, s2)

    out = pl.pallas_call(
        _norm_body,
        grid=(nb,),
        in_specs=[
            pl.BlockSpec((BLK, d), lambda i: (i, 0)),
            pl.BlockSpec((1, 1, BLK), lambda i: (i, 0, 0)),
            pl.BlockSpec((G, 2 * d), lambda i: (0, 0)),
        ],
        out_specs=pl.BlockSpec((BLK, d), lambda i: (i, 0)),
        out_shape=jax.ShapeDtypeStruct((n, d), jnp.float32),
    )(node_emb, batch3, rb)
    return out


# speedup vs baseline: 2.7824x; 1.0038x over previous
import jax
import jax.numpy as jnp
from jax import lax
from jax.experimental import pallas as pl
from jax.experimental.pallas import tpu as pltpu
from jax.experimental.pallas import tpu_sc as plsc

N = 50000
D = 256
G = 256
BLK = 2000
SC_LO = 48000
NTC = SC_LO // BLK

NC = 2
NS = 16
DC = D // NC
NG16 = DC // 16
SCN = N - SC_LO
SBR = 80
CHUNK = (SCN // NS + 7) // 8 * 8
NFULL = (SCN // NS - 8) // SBR


def _sc_stats_body(x_hbm, batch_hbm, o1, o2,
                   p_sh, s1v, s2v, xv, idxv, semx):
    cid = lax.axis_index("c")
    sid = lax.axis_index("s")
    ccol = cid * DC
    zeros16 = jnp.zeros((16,), jnp.float32)

    def _zero(r, _):
        for c in range(NG16):
            s1v[r, pl.ds(c * 16, 16)] = zeros16
            s2v[r, pl.ds(c * 16, 16)] = zeros16
        return 0

    lax.fori_loop(0, G, _zero, 0)

    start = SC_LO + sid * SCN // NS // 8 * 8
    end = SC_LO + (sid + 1) * SCN // NS // 8 * 8
    nrows = end - start

    pltpu.sync_copy(batch_hbm.at[pl.ds(start, CHUNK)], idxv)

    def _start_x(k, b):
        return pltpu.async_copy(
            x_hbm.at[pl.ds(start + k * SBR, SBR), pl.ds(ccol, DC)],
            xv.at[b], semx.at[b])

    def _process(b, ioff, lo):
        def _row16(i, _):
            idvec = idxv[pl.ds(ioff + i * 16, 16)]
            g0 = idvec[0]
            g15 = idvec[15]
            a1 = [jnp.zeros((16,), jnp.float32)] * NG16
            a2 = [jnp.zeros((16,), jnp.float32)] * NG16
            for l in range(16):
                r = i * 16 + l
                ma = ((r >= lo) & (idvec[l] == g0)).astype(jnp.float32)
                for c in range(NG16):
                    v = xv[b, r, pl.ds(c * 16, 16)]
                    a1[c] = a1[c] + v * ma
                    a2[c] = a2[c] + (v * v) * ma
            for c in range(NG16):
                plsc.addupdate(s1v.at[g0, pl.ds(c * 16, 16)], a1[c])
                plsc.addupdate(s2v.at[g0, pl.ds(c * 16, 16)], a2[c])

            @pl.when(g15 != g0)
            def _slow():
                idv = idxv[pl.ds(ioff + i * 16, 16)]
                gg0 = idv[0]
                for l in range(16):
                    r = i * 16 + l
                    g = idv[l]
                    mb = ((r >= lo) & (g != gg0)).astype(jnp.float32)
                    for c in range(NG16):
                        v = xv[b, r, pl.ds(c * 16, 16)] * mb
                        plsc.addupdate(s1v.at[g, pl.ds(c * 16, 16)], v)
                        plsc.addupdate(s2v.at[g, pl.ds(c * 16, 16)], v * v)

            return 0

        lax.fori_loop(0, SBR // 16, _row16, 0)

    _start_x(0, 0)
    if NFULL >= 2:
        _start_x(1, 1)

    @pl.loop(0, NFULL // 2 * 2, step=2)
    def _ring(k):
        for b in range(2):
            kk = k + b
            pltpu.make_async_copy(
                x_hbm.at[pl.ds(start + kk * SBR, SBR), pl.ds(ccol, DC)],
                xv.at[b], semx.at[b]).wait()
            _process(b, kk * SBR, 0)

            @pl.when(kk + 2 < NFULL)
            def _next():
                _start_x(kk + 2, b)

    if NFULL % 2:
        kk = NFULL - 1
        pltpu.make_async_copy(
            x_hbm.at[pl.ds(start + kk * SBR, SBR), pl.ds(ccol, DC)],
            xv.at[0], semx.at[0]).wait()
        _process(0, kk * SBR, 0)

    rem = nrows - NFULL * SBR
    pltpu.sync_copy(x_hbm.at[pl.ds(end - SBR, SBR), pl.ds(ccol, DC)], xv.at[0])
    _process(0, nrows - SBR, SBR - rem)

    GH = G // 2
    STRIP = GH // NS

    def _phase(src_v, glo, out_hbm):
        pltpu.sync_copy(src_v.at[pl.ds(glo, GH)], p_sh.at[sid])
        plsc.subcore_barrier()
        rlo = sid * STRIP
        pltpu.sync_copy(p_sh.at[0, pl.ds(rlo, STRIP)],
                        xv.at[0, pl.ds(0, STRIP)])

        def _racc(j, _):
            pltpu.sync_copy(p_sh.at[j, pl.ds(rlo, STRIP)],
                            xv.at[0, pl.ds(STRIP, STRIP)])

            def _radd(r, _):
                for c in range(NG16):
                    plsc.addupdate(xv.at[0, r, pl.ds(c * 16, 16)],
                                   xv[0, STRIP + r, pl.ds(c * 16, 16)])
                return 0

            lax.fori_loop(0, STRIP, _radd, 0)
            return 0

        lax.fori_loop(1, NS, _racc, 0)
        pltpu.sync_copy(xv.at[0, pl.ds(0, STRIP)],
                        out_hbm.at[pl.ds(glo + rlo, STRIP), pl.ds(ccol, DC)])
        plsc.subcore_barrier()

    _phase(s1v, 0, o1)
    _phase(s1v, GH, o1)
    _phase(s2v, 0, o2)
    _phase(s2v, GH, o2)


def _tcstats_body(x_ref, batch_ref, acc_out_ref, acc_ref):
    i = pl.program_id(0)
    nb = pl.num_programs(0)

    @pl.when(i == 0)
    def _init():
        acc_ref[...] = jnp.zeros_like(acc_ref)

    bb = batch_ref[0, 0, :]
    onehot = (bb[:, None] == lax.broadcasted_iota(jnp.int32, (BLK, G), 1)
              ).astype(jnp.bfloat16)
    x = x_ref[...].astype(jnp.bfloat16)
    ones = jnp.ones((BLK, 8), jnp.bfloat16)
    xcat = jnp.concatenate([x, x * x, ones], axis=1)
    acc_ref[...] += lax.dot_general(
        onehot, xcat, (((0,), (0,)), ((), ())),
        preferred_element_type=jnp.float32)

    @pl.when(i == nb - 1)
    def _fin():
        acc_out_ref[...] = acc_ref[...]


def _finalize_body(batch_ref, tc_ref, o1_ref, o2_ref, w_ref, b_ref, s_ref,
                   rb_ref, cnt_ref):
    i = pl.program_id(0)
    nb = pl.num_programs(0)

    @pl.when(i == 0)
    def _init():
        cnt_ref[...] = jnp.zeros_like(cnt_ref)

    bb = batch_ref[0, 0, :]
    onehot = (bb[:, None] == lax.broadcasted_iota(jnp.int32, (BLK, G), 1)
              ).astype(jnp.float32)
    cnt_ref[...] += jnp.sum(onehot, axis=0)[None, :]

    @pl.when(i == nb - 1)
    def _fin():
        s1 = tc_ref[:, :D] + o1_ref[...]
        s2 = tc_ref[:, D:2 * D] + o2_ref[...]
        cnt = cnt_ref[0, :][:, None] + tc_ref[:, 2 * D:2 * D + 1]
        denom = jnp.maximum(cnt, 1.0)
        a = (s1 / denom) * s_ref[...]
        var = (s2 - 2.0 * a * s1 + cnt * a * a) / denom
        r = w_ref[...] * lax.rsqrt(var + 1e-8)
        bp = b_ref[...] - a * r
        rb_ref[...] = jnp.concatenate([r, bp], axis=1)


def _norm_body(x_ref, batch_ref, rb_ref, out_ref):
    bb = batch_ref[0, 0, :]
    onehot = (bb[:, None] == lax.broadcasted_iota(jnp.int32, (BLK, G), 1)
              ).astype(jnp.bfloat16)
    g = lax.dot_general(onehot, rb_ref[...].astype(jnp.bfloat16),
                        (((1,), (0,)), ((), ())),
                        preferred_element_type=jnp.float32)
    x = x_ref[...]
    out_ref[...] = x * g[:, :D] + g[:, D:]


@jax.jit
def kernel(node_emb, weight, bias, scale, batch):
    n, d = node_emb.shape
    nb = n // BLK
    batch_i = batch.astype(jnp.int32)
    batch3 = batch_i.reshape(nb, 1, BLK)

    mesh = plsc.VectorSubcoreMesh(core_axis_name="c", subcore_axis_name="s")
    o1, o2 = pl.kernel(
        _sc_stats_body,
        out_type=(
            jax.ShapeDtypeStruct((G, D), jnp.float32),
            jax.ShapeDtypeStruct((G, D), jnp.float32),
        ),
        mesh=mesh,
        scratch_types=[
            pltpu.VMEM_SHARED((NS, G // 2, DC), jnp.float32),
            pltpu.VMEM((G, DC), jnp.float32),
            pltpu.VMEM((G, DC), jnp.float32),
            pltpu.VMEM((2, SBR, DC), jnp.float32),
            pltpu.VMEM((CHUNK,), jnp.int32),
            pltpu.SemaphoreType.DMA((2,)),
        ],
    )(node_emb, batch_i)

    tc_acc = pl.pallas_call(
        _tcstats_body,
        grid=(NTC,),
        in_specs=[
            pl.BlockSpec((BLK, d), lambda i: (i, 0)),
            pl.BlockSpec((1, 1, BLK), lambda i: (i, 0, 0)),
        ],
        out_specs=pl.BlockSpec((G, 2 * d + 8), lambda i: (0, 0)),
        out_shape=jax.ShapeDtypeStruct((G, 2 * d + 8), jnp.float32),
        scratch_shapes=[pltpu.VMEM((G, 2 * d + 8), jnp.float32)],
    )(node_emb, batch3)

    w2 = weight.reshape(1, d)
    b2 = bias.reshape(1, d)
    s2 = scale.reshape(1, d)
    rb = pl.pallas_call(
        _finalize_body,
        grid=(nb - NTC,),
        in_specs=[
            pl.BlockSpec((1, 1, BLK), lambda i: (NTC + i, 0, 0)),
            pl.BlockSpec((G, 2 * d + 8), lambda i: (0, 0)),
            pl.BlockSpec((G, d), lambda i: (0, 0)),
            pl.BlockSpec((G, d), lambda i: (0, 0)),
            pl.BlockSpec((1, d), lambda i: (0, 0)),
            pl.BlockSpec((1, d), lambda i: (0, 0)),
            pl.BlockSpec((1, d), lambda i: (0, 0)),
        ],
        out_specs=pl.BlockSpec((G, 2 * d), lambda i: (0, 0)),
        out_shape=jax.ShapeDtypeStruct((G, 2 * d), jnp.float32),
        scratch_shapes=[pltpu.VMEM((1, G), jnp.float32)],
    )(batch3, tc_acc, o1, o2, w2, b2, s2)

    out = pl.pallas_call(
        _norm_body,
        grid=(nb,),
        in_specs=[
            pl.BlockSpec((BLK, d), lambda i: (i, 0)),
            pl.BlockSpec((1, 1, BLK), lambda i: (i, 0, 0)),
            pl.BlockSpec((G, 2 * d), lambda i: (0, 0)),
        ],
        out_specs=pl.BlockSpec((BLK, d), lambda i: (i, 0)),
        out_shape=jax.ShapeDtypeStruct((n, d), jnp.float32),
    )(node_emb, batch3, rb)
    return out
